# baseline (speedup 1.0000x reference)
import jax
import jax.numpy as jnp
from jax import lax
from jax.experimental import pallas as pl
from jax.experimental.pallas import tpu as pltpu
from jax.experimental.pallas import tpu_sc as plsc

_BATCH = 16384
_NF = 26
_NEMB = 13
_VOCAB = 500
_TROW = 5
_DIM = 4
_OUT = 64
_NW = 32
_CHUNK = _BATCH // _NW
_HROWS = 256
_PROWS = 128
_NPASS = _CHUNK // _PROWS
_PGROUPS = _PROWS // 16
_LANES = 16
_TS = 513
_OS = 129


def _sc_body(tab_hbm, us_hbm, out_hbm, tab_v, us_half, us_t, out_t, out_v,
             sem_t):
    wid = lax.axis_index("s") * 2 + lax.axis_index("c")
    chunk0 = wid * _CHUNK
    ct = pltpu.async_copy(tab_hbm, tab_v, sem_t)
    iotav = lax.iota(jnp.int32, _LANES)

    for h in range(_CHUNK // _HROWS):
        pltpu.sync_copy(us_hbm.at[pl.ds(chunk0 + h * _HROWS, _HROWS)],
                        us_half)

        @pl.loop(0, _HROWS, unroll=8)
        def _row(b, h=h):
            grow = b + (h * _HROWS)
            lo = us_half[b, pl.ds(0, _LANES)]
            hi = us_half[b, pl.ds(_NF - _LANES - 1, _LANES)]
            plsc.store_scatter(us_t, [iotav * _TS + grow], lo)
            plsc.store_scatter(
                us_t, [(iotav + (_NF - _LANES - 1)) * _TS + grow], hi)

    ct.wait()

    for p in range(_NPASS):

        @pl.loop(0, _PGROUPS, unroll=2)
        def _group(g, p=p):
            base = g * _LANES + (p * _PROWS)
            lbase = g * _LANES
            for f in range(_NEMB):
                iv = us_t[pl.ds(f * _TS + base, _LANES)]
                addr = iv * _TROW + (f * _VOCAB * _TROW)
                for d in range(_DIM):
                    v = plsc.load_gather(tab_v, [addr + d])
                    out_t[pl.ds((4 * f + d) * _OS + lbase, _LANES)] = v
            for f in range(_NEMB, _NF - 1):
                iv = us_t[pl.ds(f * _TS + base, _LANES)]
                out_t[pl.ds((f + 39) * _OS + lbase, _LANES)] = (
                    iv.astype(jnp.float32))

        @pl.loop(0, _PROWS, unroll=4)
        def _orow(r):
            for k in range(_OUT // _LANES):
                cg = (iotav + k * _LANES) * _OS + r
                out_v[r, pl.ds(k * _LANES, _LANES)] = (
                    plsc.load_gather(out_t, [cg]))

        pltpu.sync_copy(out_v,
                        out_hbm.at[pl.ds(chunk0 + p * _PROWS, _PROWS)])


@jax.jit
def kernel(user_sparse, tables):
    mesh = plsc.VectorSubcoreMesh(core_axis_name="c", subcore_axis_name="s")
    run = pl.kernel(
        _sc_body,
        mesh=mesh,
        compiler_params=pltpu.CompilerParams(needs_layout_passes=False),
        out_type=jax.ShapeDtypeStruct((_BATCH, _OUT), jnp.float32),
        scratch_types=[
            pltpu.VMEM((_NEMB * _VOCAB * _TROW,), jnp.float32),
            pltpu.VMEM((_HROWS, _NF), jnp.int32),
            pltpu.VMEM((2 * _LANES * _TS,), jnp.int32),
            pltpu.VMEM((_OUT * _OS,), jnp.float32),
            pltpu.VMEM((_PROWS, _OUT), jnp.float32),
            pltpu.SemaphoreType.DMA,
        ],
    )
    tab_pad = jnp.pad(tables, ((0, 0), (0, 0), (0, _TROW - _DIM)))
    return run(tab_pad.reshape(-1), user_sparse)


# speedup vs baseline: 1.1054x; 1.1054x over previous
import functools

import jax
import jax.numpy as jnp
from jax import lax
from jax.experimental import pallas as pl
from jax.experimental.pallas import tpu as pltpu
from jax.experimental.pallas import tpu_sc as plsc

_BATCH = 16384
_NF = 26
_NEMB = 13
_VOCAB = 500
_DIM = 4
_OUT = 64
_NW = 32
_CHUNK = _BATCH // _NW
_PROWS = 128
_NPASS = _CHUNK // _PROWS
_PGROUPS = _PROWS // 16
_LANES = 16


def _sc_body(tab_hbm, us_hbm, out_hbm, tab_v, us_v, out_v0, out_v1,
             sem_t, sem_u, sem_o0, sem_o1):
    wid = lax.axis_index("s") * 2 + lax.axis_index("c")
    chunk0 = wid * _CHUNK
    ct = pltpu.async_copy(tab_hbm, tab_v, sem_t)
    cu = pltpu.async_copy(us_hbm.at[pl.ds(chunk0, _CHUNK)], us_v, sem_u)
    ct.wait()
    cu.wait()

    out_bufs = (out_v0, out_v1)
    out_sems = (sem_o0, sem_o1)
    copies = [None, None]
    for p in range(_NPASS):
        buf = out_bufs[p % 2]

        @pl.loop(0, _PGROUPS, unroll=2)
        def _group(g, buf=buf, p=p):
            lrows = g * _LANES + lax.iota(jnp.int32, _LANES)
            grows = lrows + (p * _PROWS)
            idxs = []
            for f in range(_NF - 1):
                fv = jnp.full((_LANES,), f, jnp.int32)
                idxs.append(plsc.load_gather(us_v, [grows, fv]))
            vals = []
            for f in range(_NEMB):
                addr = idxs[f] * _DIM + (f * _VOCAB * _DIM)
                for d in range(_DIM):
                    vals.append(plsc.load_gather(tab_v, [addr + d]))
            for c in range(_NEMB * _DIM):
                cv = jnp.full((_LANES,), c, jnp.int32)
                plsc.store_scatter(buf, [lrows, cv], vals[c])
            for f in range(_NEMB, _NF - 1):
                cv = jnp.full((_LANES,), f + 39, jnp.int32)
                plsc.store_scatter(buf, [lrows, cv],
                                   idxs[f].astype(jnp.float32))

        pltpu.sync_copy(buf, out_hbm.at[pl.ds(chunk0 + p * _PROWS, _PROWS)])


@jax.jit
def kernel(user_sparse, tables):
    mesh = plsc.VectorSubcoreMesh(core_axis_name="c", subcore_axis_name="s")
    run = functools.partial(
        pl.kernel,
        mesh=mesh,
        compiler_params=pltpu.CompilerParams(needs_layout_passes=False),
        out_type=jax.ShapeDtypeStruct((_BATCH, _OUT), jnp.float32),
        scratch_types=[
            pltpu.VMEM((_NEMB * _VOCAB * _DIM,), jnp.float32),
            pltpu.VMEM((_CHUNK, _NF), jnp.int32),
            pltpu.VMEM((_PROWS, _OUT), jnp.float32),
            pltpu.VMEM((_PROWS, _OUT), jnp.float32),
            pltpu.SemaphoreType.DMA,
            pltpu.SemaphoreType.DMA,
            pltpu.SemaphoreType.DMA,
            pltpu.SemaphoreType.DMA,
        ],
    )(_sc_body)
    return run(tables.reshape(-1), user_sparse)
